# Initial kernel scaffold; baseline (speedup 1.0000x reference)
import jax
import jax.numpy as jnp
from jax.experimental import pallas as pl


def kernel(x, edge_index, lap_pe, W1, b1, W2, b2, Wmu, bmu, Wlv, blv):
    raise NotImplementedError("write your pallas kernel here")


# speedup vs baseline: 10.7658x; 10.7658x over previous
import functools

import jax
import jax.numpy as jnp
from jax import lax
from jax.experimental import pallas as pl
from jax.experimental.pallas import tpu as pltpu
from jax.experimental.pallas import tpu_sc as plsc

N = 10000
E = 320000
N_PAD = 10240
DUMMY = N
NW = 32
EPW = E // NW
CHUNK = 80
NCHUNK = EPW // CHUNK
RPT = N_PAD // 16
D = 128
BLK = 1024
GRID = N_PAD // BLK


def _mesh():
    return plsc.VectorSubcoreMesh(core_axis_name="c", subcore_axis_name="s")


_SC_PARAMS = pltpu.CompilerParams(needs_layout_passes=False)


@functools.partial(
    pl.kernel,
    out_type=jax.ShapeDtypeStruct((NW, N_PAD), jnp.float32),
    mesh=_mesh(),
    compiler_params=_SC_PARAMS,
    scratch_types=[
        pltpu.VMEM((NCHUNK, CHUNK), jnp.int32),
        pltpu.VMEM((NCHUNK, CHUNK), jnp.int32),
        pltpu.VMEM((N_PAD,), jnp.float32),
    ],
)
def _deg(e4, out, src_v, dst_v, hist):
    cid = lax.axis_index("c")
    sid = lax.axis_index("s")
    wid = sid * 2 + cid
    pltpu.sync_copy(e4.at[0, wid], src_v)
    pltpu.sync_copy(e4.at[1, wid], dst_v)

    @pl.loop(0, N_PAD // 16)
    def _zero(i):
        hist[pl.ds(i * 16, 16)] = jnp.zeros((16,), jnp.float32)

    ones = jnp.ones((16,), jnp.float32)

    @pl.loop(0, NCHUNK)
    def _hist(r):
        for j in range(CHUNK // 16):
            s = src_v[r, pl.ds(j * 16, 16)]
            d = dst_v[r, pl.ds(j * 16, 16)]
            sp = jnp.where(s != d, s, DUMMY)
            plsc.addupdate_scatter(hist, [sp], ones)

    pltpu.sync_copy(hist, out.at[wid])


@functools.partial(
    pl.kernel,
    out_type=jax.ShapeDtypeStruct((2, N_PAD, D), jnp.float32),
    mesh=_mesh(),
    compiler_params=_SC_PARAMS,
    scratch_types=[
        pltpu.VMEM((NCHUNK, CHUNK), jnp.int32),
        pltpu.VMEM((NCHUNK, CHUNK), jnp.int32),
        pltpu.VMEM((CHUNK, D), jnp.float32),
        pltpu.VMEM((32, D), jnp.float32),
        pltpu.VMEM_SHARED((N_PAD, D), jnp.float32),
        pltpu.SemaphoreType.DMA,
    ],
)
def _lhat(e4, vs, out, src_v, dst_v, rows_v, zbuf, acc, sem):
    cid = lax.axis_index("c")
    sid = lax.axis_index("s")
    wid = sid * 2 + cid
    pltpu.sync_copy(e4.at[0, wid], src_v)
    pltpu.sync_copy(e4.at[1, wid], dst_v)

    @pl.loop(0, 32)
    def _zb(r):
        for j in range(D // 16):
            zbuf[r, pl.ds(j * 16, 16)] = jnp.zeros((16,), jnp.float32)

    @pl.loop(0, RPT // 32)
    def _za(t):
        pltpu.sync_copy(zbuf, acc.at[pl.ds(sid * RPT + t * 32, 32)])

    @pl.loop(0, NCHUNK)
    def _fix(r):
        for j in range(CHUNK // 16):
            s = src_v[r, pl.ds(j * 16, 16)]
            dd = dst_v[r, pl.ds(j * 16, 16)]
            dst_v[r, pl.ds(j * 16, 16)] = jnp.where(s != dd, dd, DUMMY)

    plsc.subcore_barrier()

    @pl.loop(0, NCHUNK)
    def _edges(c):
        pltpu.async_copy(vs.at[src_v.at[c]], rows_v, sem).wait()
        pltpu.sync_copy(rows_v, acc.at[dst_v.at[c]], add=True)

    plsc.subcore_barrier()
    pltpu.sync_copy(acc.at[pl.ds(sid * RPT, RPT)],
                    out.at[cid, pl.ds(sid * RPT, RPT)])


def _dinv_body(h_ref, o_ref):
    pid = pl.program_id(0)
    s = jnp.sum(h_ref[...], axis=0)
    row = lax.broadcasted_iota(jnp.int32, (BLK,), 0) + pid * BLK
    o_ref[...] = jnp.where((s > 0.0) & (row < N), lax.rsqrt(s), 0.0)[:, None]


def _dinv_call(hists):
    return pl.pallas_call(
        _dinv_body,
        grid=(GRID,),
        in_specs=[pl.BlockSpec((NW, BLK), lambda i: (0, i))],
        out_specs=pl.BlockSpec((BLK, 1), lambda i: (i, 0)),
        out_shape=jax.ShapeDtypeStruct((N_PAD, 1), jnp.float32),
    )(hists)


def _row(i):
    return (i, 0)


def _col(k):
    return lambda i: (i, k)


def _start_body(h_ref, w_ref, dv_ref, c_ref, vs_ref):
    c = jnp.dot(h_ref[...], w_ref[...], preferred_element_type=jnp.float32)
    c_ref[...] = c
    vs_ref[...] = c[:, 3 * D:] * dv_ref[...]


def _start_call(h, wcat, dinv):
    d_in = h.shape[1]
    return pl.pallas_call(
        _start_body,
        grid=(GRID,),
        in_specs=[
            pl.BlockSpec((BLK, d_in), _row),
            pl.BlockSpec((d_in, 4 * D), lambda i: (0, 0)),
            pl.BlockSpec((BLK, 1), _row),
        ],
        out_specs=[
            pl.BlockSpec((BLK, 4 * D), _row),
            pl.BlockSpec((BLK, D), _row),
        ],
        out_shape=[
            jax.ShapeDtypeStruct((N_PAD, 4 * D), jnp.float32),
            jax.ShapeDtypeStruct((N_PAD, D), jnp.float32),
        ],
    )(h, wcat, dinv)


def _s1_body(a_ref, dv_ref, c2_ref, b2_ref, vs_ref):
    dv = dv_ref[...]
    b2 = c2_ref[...] - 2.0 * (a_ref[0] + a_ref[1]) * dv
    b2_ref[...] = b2
    vs_ref[...] = b2 * dv


def _s1_call(acc, dinv, c):
    return pl.pallas_call(
        _s1_body,
        grid=(GRID,),
        in_specs=[
            pl.BlockSpec((2, BLK, D), lambda i: (0, i, 0)),
            pl.BlockSpec((BLK, 1), _row),
            pl.BlockSpec((BLK, D), _col(2)),
        ],
        out_specs=[
            pl.BlockSpec((BLK, D), _row),
            pl.BlockSpec((BLK, D), _row),
        ],
        out_shape=[
            jax.ShapeDtypeStruct((N_PAD, D), jnp.float32),
            jax.ShapeDtypeStruct((N_PAD, D), jnp.float32),
        ],
    )(acc, dinv, c)


def _s2_body(a_ref, dv_ref, c1_ref, c3_ref, vs_ref):
    dv = dv_ref[...]
    b1 = c1_ref[...] - 2.0 * (a_ref[0] + a_ref[1]) * dv - c3_ref[...]
    vs_ref[...] = b1 * dv


def _s2_call(acc, dinv, c):
    return pl.pallas_call(
        _s2_body,
        grid=(GRID,),
        in_specs=[
            pl.BlockSpec((2, BLK, D), lambda i: (0, i, 0)),
            pl.BlockSpec((BLK, 1), _row),
            pl.BlockSpec((BLK, D), _col(1)),
            pl.BlockSpec((BLK, D), _col(3)),
        ],
        out_specs=pl.BlockSpec((BLK, D), _row),
        out_shape=jax.ShapeDtypeStruct((N_PAD, D), jnp.float32),
    )(acc, dinv, c, c)


def _end_body(a_ref, dv_ref, c0_ref, b2_ref, b_ref, h_ref):
    out = c0_ref[...] - (a_ref[0] + a_ref[1]) * dv_ref[...] - b2_ref[...]
    h_ref[...] = jnp.maximum(out + b_ref[...], 0.0)


def _end_call(acc, dinv, c, b2, bias):
    return pl.pallas_call(
        _end_body,
        grid=(GRID,),
        in_specs=[
            pl.BlockSpec((2, BLK, D), lambda i: (0, i, 0)),
            pl.BlockSpec((BLK, 1), _row),
            pl.BlockSpec((BLK, D), _col(0)),
            pl.BlockSpec((BLK, D), _row),
            pl.BlockSpec((1, D), lambda i: (0, 0)),
        ],
        out_specs=pl.BlockSpec((BLK, D), _row),
        out_shape=jax.ShapeDtypeStruct((N_PAD, D), jnp.float32),
    )(acc, dinv, c, b2, bias)


def _pool_body(h_ref, wmu_ref, bmu_ref, wlv_ref, blv_ref, mu_ref, lv_ref):
    g = jnp.sum(h_ref[...], axis=0, keepdims=True) * (1.0 / N)
    mu_ref[...] = jnp.dot(g, wmu_ref[...],
                          preferred_element_type=jnp.float32) + bmu_ref[...]
    lv_ref[...] = jnp.dot(g, wlv_ref[...],
                          preferred_element_type=jnp.float32) + blv_ref[...]


def _pool_call(h, wmu, bmu, wlv, blv):
    lat = wmu.shape[1]
    return pl.pallas_call(
        _pool_body,
        out_shape=[
            jax.ShapeDtypeStruct((1, lat), jnp.float32),
            jax.ShapeDtypeStruct((1, lat), jnp.float32),
        ],
    )(h, wmu, bmu, wlv, blv)


def kernel(x, edge_index, lap_pe, W1, b1, W2, b2, Wmu, bmu, Wlv, blv):
    e4 = edge_index.astype(jnp.int32).reshape(2, NW, NCHUNK, CHUNK)
    h = jnp.concatenate([x, lap_pe], axis=1)
    h = jnp.pad(h, ((0, N_PAD - N), (0, 0)))

    hists = _deg(e4)
    dinv = _dinv_call(hists)

    for W, bias in ((W1, b1), (W2, b2)):
        wcat = W.transpose(1, 0, 2).reshape(W.shape[1], 4 * D)
        c, vs = _start_call(h, wcat, dinv)
        a = _lhat(e4, vs)
        b2v, vs = _s1_call(a, dinv, c)
        a = _lhat(e4, vs)
        vs = _s2_call(a, dinv, c)
        a = _lhat(e4, vs)
        h = _end_call(a, dinv, c, b2v, bias.reshape(1, -1))

    mu, lv = _pool_call(h[:N], Wmu, bmu.reshape(1, -1), Wlv, blv.reshape(1, -1))
    return (mu, lv)
